# baseline (speedup 1.0000x reference)
import functools

import jax
import jax.numpy as jnp
from jax import lax
from jax.experimental import pallas as pl
from jax.experimental.pallas import tpu as pltpu
from jax.experimental.pallas import tpu_sc as plsc

_CH = 40
_NB = 5


def _node_matmul_kernel(x_ref, wst_ref, wdt_ref, b_ref, ys_ref, yd_ref):
    xv = x_ref[...]
    ys_ref[...] = jnp.dot(xv, wst_ref[...], preferred_element_type=jnp.float32)
    yd_ref[...] = (
        jnp.dot(xv, wdt_ref[...], preferred_element_type=jnp.float32) + b_ref[...]
    )


def _edge_gather_kernel(
    nc,
    nw,
    ys_hbm,
    yd_hbm,
    src_hbm,
    dst_hbm,
    out_hbm,
    src_all,
    dst_all,
    rows,
    ys_sp,
    sem_r,
):
    sid = lax.axis_index("s")
    wid = sid * nc + lax.axis_index("c")
    per_w = out_hbm.shape[0] // nw
    base = wid * per_w
    n_chunks = per_w // _CH
    n_outer = n_chunks // _NB

    n_nodes = ys_hbm.shape[0]
    rows_per_tile = n_nodes // 10

    @pl.when(sid < 10)
    def _():
        pltpu.sync_copy(
            ys_hbm.at[pl.ds(sid * rows_per_tile, rows_per_tile)],
            ys_sp.at[pl.ds(sid * rows_per_tile, rows_per_tile)],
        )

    pltpu.sync_copy(src_hbm.at[pl.ds(base, per_w)], src_all)
    pltpu.sync_copy(dst_hbm.at[pl.ds(base, per_w)], dst_all)
    plsc.subcore_barrier()

    def outer(g, carry):
        descs = [None] * _NB
        for b in range(_NB):
            off = (g * _NB + b) * _CH

            @pl.when(g > 0)
            def _():
                pltpu.make_async_copy(
                    out_hbm.at[pl.ds(base + off, _CH)], rows.at[b], sem_r.at[b]
                ).wait()

            descs[b] = pltpu.async_copy(
                yd_hbm.at[dst_all.at[pl.ds(off, _CH)]], rows.at[b], sem_r.at[b]
            )
        for b in range(_NB):
            off = (g * _NB + b) * _CH
            descs[b].wait()
            descs[b] = pltpu.async_copy(
                ys_sp.at[src_all.at[pl.ds(off, _CH)]],
                rows.at[b],
                sem_r.at[b],
                add=True,
            )
        for b in range(_NB):
            off = (g * _NB + b) * _CH
            descs[b].wait()
            pltpu.async_copy(rows.at[b], out_hbm.at[pl.ds(base + off, _CH)], sem_r.at[b])
        return carry

    lax.fori_loop(0, n_outer, outer, 0)
    for b in range(_NB):
        pltpu.make_async_copy(
            out_hbm.at[pl.ds(base, _CH)], rows.at[b], sem_r.at[b]
        ).wait()


def kernel(x, edge_index, W, b):
    n_nodes, d_in = x.shape
    d_out = W.shape[0]
    n_edges = edge_index.shape[1]

    wst = W[:, :d_in].T
    wdt = W[:, d_in:].T
    ys, yd = pl.pallas_call(
        _node_matmul_kernel,
        out_shape=[
            jax.ShapeDtypeStruct((n_nodes, d_out), jnp.float32),
            jax.ShapeDtypeStruct((n_nodes, d_out), jnp.float32),
        ],
    )(x, wst, wdt, b.reshape(1, d_out))

    src = edge_index[0].astype(jnp.int32)
    dst = edge_index[1].astype(jnp.int32)

    mesh = plsc.VectorSubcoreMesh(core_axis_name="c", subcore_axis_name="s")
    nw = mesh.num_cores * mesh.num_subcores
    out = pl.kernel(
        functools.partial(_edge_gather_kernel, mesh.num_cores, nw),
        out_type=jax.ShapeDtypeStruct((n_edges, d_out), jnp.float32),
        mesh=mesh,
        scratch_types=[
            pltpu.VMEM((n_edges // nw,), jnp.int32),
            pltpu.VMEM((n_edges // nw,), jnp.int32),
            pltpu.VMEM((_NB, _CH, d_out), jnp.float32),
            pltpu.VMEM_SHARED((n_nodes, d_out), jnp.float32),
            pltpu.SemaphoreType.DMA((_NB,)),
        ],
    )(ys, yd, src, dst)
    return out


# speedup vs baseline: 1.0523x; 1.0523x over previous
import functools

import jax
import jax.numpy as jnp
from jax import lax
from jax.experimental import pallas as pl
from jax.experimental.pallas import tpu as pltpu
from jax.experimental.pallas import tpu_sc as plsc

_CH = 40
_NB = 5


def _node_matmul_kernel(x_ref, wst_ref, wdt_ref, b_ref, ys_ref, yd_ref):
    xv = x_ref[...]
    ys_ref[...] = jnp.dot(xv, wst_ref[...], preferred_element_type=jnp.float32)
    yd_ref[...] = (
        jnp.dot(xv, wdt_ref[...], preferred_element_type=jnp.float32) + b_ref[...]
    )


def _edge_gather_kernel(
    nc,
    nw,
    ys_hbm,
    yd_hbm,
    src_hbm,
    dst_hbm,
    out_hbm,
    src_all,
    dst_all,
    rows,
    ys_sp,
    sem_r,
):
    sid = lax.axis_index("s")
    wid = sid * nc + lax.axis_index("c")
    per_w = out_hbm.shape[0] // nw
    base = wid * per_w
    n_chunks = per_w // _CH
    n_outer = n_chunks // _NB

    n_nodes = ys_hbm.shape[0]
    rows_per_tile = n_nodes // 10

    @pl.when(sid < 10)
    def _():
        pltpu.sync_copy(
            ys_hbm.at[pl.ds(sid * rows_per_tile, rows_per_tile)],
            ys_sp.at[pl.ds(sid * rows_per_tile, rows_per_tile)],
        )

    pltpu.sync_copy(src_hbm.at[pl.ds(base, per_w)], src_all)
    pltpu.sync_copy(dst_hbm.at[pl.ds(base, per_w)], dst_all)
    plsc.subcore_barrier()

    def outer(g, carry):
        descs = [None] * _NB
        for b in range(_NB):
            off = (g * _NB + b) * _CH

            @pl.when(g > 0)
            def _():
                pltpu.make_async_copy(
                    out_hbm.at[pl.ds(base + off, _CH)], rows.at[b], sem_r.at[b]
                ).wait()

            descs[b] = pltpu.async_copy(
                ys_sp.at[src_all.at[pl.ds(off, _CH)]], rows.at[b], sem_r.at[b]
            )
        for b in range(_NB):
            off = (g * _NB + b) * _CH
            descs[b].wait()
            descs[b] = pltpu.async_copy(
                yd_hbm.at[dst_all.at[pl.ds(off, _CH)]],
                rows.at[b],
                sem_r.at[b],
                add=True,
            )
        for b in range(_NB):
            off = (g * _NB + b) * _CH
            descs[b].wait()
            pltpu.async_copy(rows.at[b], out_hbm.at[pl.ds(base + off, _CH)], sem_r.at[b])
        return carry

    lax.fori_loop(0, n_outer, outer, 0)
    for b in range(_NB):
        pltpu.make_async_copy(
            out_hbm.at[pl.ds(base, _CH)], rows.at[b], sem_r.at[b]
        ).wait()


def kernel(x, edge_index, W, b):
    n_nodes, d_in = x.shape
    d_out = W.shape[0]
    n_edges = edge_index.shape[1]

    wst = W[:, :d_in].T
    wdt = W[:, d_in:].T
    ys, yd = pl.pallas_call(
        _node_matmul_kernel,
        out_shape=[
            jax.ShapeDtypeStruct((n_nodes, d_out), jnp.float32),
            jax.ShapeDtypeStruct((n_nodes, d_out), jnp.float32),
        ],
    )(x, wst, wdt, b.reshape(1, d_out))

    src = edge_index[0].astype(jnp.int32)
    dst = edge_index[1].astype(jnp.int32)

    mesh = plsc.VectorSubcoreMesh(core_axis_name="c", subcore_axis_name="s")
    nw = mesh.num_cores * mesh.num_subcores
    out = pl.kernel(
        functools.partial(_edge_gather_kernel, mesh.num_cores, nw),
        out_type=jax.ShapeDtypeStruct((n_edges, d_out), jnp.float32),
        mesh=mesh,
        scratch_types=[
            pltpu.VMEM((n_edges // nw,), jnp.int32),
            pltpu.VMEM((n_edges // nw,), jnp.int32),
            pltpu.VMEM((_NB, _CH, d_out), jnp.float32),
            pltpu.VMEM_SHARED((n_nodes, d_out), jnp.float32),
            pltpu.SemaphoreType.DMA((_NB,)),
        ],
    )(ys, yd, src, dst)
    return out
